# baseline (speedup 1.0000x reference)
import jax
import jax.numpy as jnp
from jax import lax
from jax.experimental import pallas as pl
from jax.experimental.pallas import tpu as pltpu
from jax.experimental.pallas import tpu_sc as plsc

_N = 20000
_IOU_THR = 0.5
_SCORE_THR = 0.1
_MAX_KEEP = 256
_NEG = -1e10

_L = 16
_NS = 16
_NPAD = 20480
_SHARD = _NPAD // _NS
_NSLICE = _SHARD // _L
_FNEG = -3.0e38


def _nms_body(x1_h, y1_h, x2_h, y2_h, s_h, out_h,
              x1_v, y1_v, x2_v, y2_v, s_v, area_v, kept_v, tab_v, comm_v,
              tbl_sh):
    wid = lax.axis_index("s")
    loff = wid * _SHARD
    iota = lax.iota(jnp.int32, _L)
    zeros_i = jnp.zeros((_L,), jnp.int32)
    zf = jnp.zeros((_L,), jnp.float32)

    pltpu.sync_copy(x1_h, x1_v)
    pltpu.sync_copy(y1_h, y1_v)
    pltpu.sync_copy(x2_h, x2_v)
    pltpu.sync_copy(y2_h, y2_v)
    pltpu.sync_copy(s_h.at[pl.ds(loff, _SHARD)], s_v)

    @plsc.parallel_loop(0, 5 * _MAX_KEEP, unroll=8)
    def _zero(j):
        kept_v[j, :] = zf

    def _top2_update(carry, vn, idxv):
        a1v, a1i, a2v, a2i = carry
        g1 = vn > a1v
        g2 = vn > a2v
        n1v = jnp.where(g1, vn, a1v)
        n1i = jnp.where(g1, idxv, a1i)
        n2v = jnp.where(g1, a1v, jnp.where(g2, vn, a2v))
        n2i = jnp.where(g1, a1i, jnp.where(g2, idxv, a2i))
        return n1v, n1i, n2v, n2i

    def _merge2(a, b):
        a1v, a1i, a2v, a2i = a
        b1v, b1i, b2v, b2i = b
        tb = (b1v > a1v) | ((b1v == a1v) & (b1i < a1i))
        t1v = jnp.where(tb, b1v, a1v)
        t1i = jnp.where(tb, b1i, a1i)
        cav = jnp.where(tb, a1v, a2v)
        cai = jnp.where(tb, a1i, a2i)
        cbv = jnp.where(tb, b2v, b1v)
        cbi = jnp.where(tb, b2i, b1i)
        t2 = (cbv > cav) | ((cbv == cav) & (cbi < cai))
        t2v = jnp.where(t2, cbv, cav)
        t2i = jnp.where(t2, cbi, cai)
        return t1v, t1i, t2v, t2i

    def _butterfly2(s):
        for sh in (8, 4, 2, 1):
            perm = iota ^ sh
            p = tuple(x.at[perm].get(mode="promise_in_bounds") for x in s)
            s = _merge2(s, p)
        return s

    def _publish(s, slot):
        m1, i1, m2, i2 = _butterfly2(s)
        row = jnp.where(iota == 0, m1,
                        jnp.where(iota == 1, plsc.bitcast(i1, jnp.float32),
                                  jnp.where(iota == 2, m2,
                                            plsc.bitcast(i2, jnp.float32))))
        comm_v[...] = row
        pltpu.sync_copy(comm_v, tbl_sh.at[slot, wid])
        plsc.subcore_barrier()

    def _box(idx_v):
        bx1 = plsc.load_gather(x1_v, [idx_v])
        by1 = plsc.load_gather(y1_v, [idx_v])
        bx2 = plsc.load_gather(x2_v, [idx_v])
        by2 = plsc.load_gather(y2_v, [idx_v])
        ba = jnp.maximum(bx2 - bx1, 0.0) * jnp.maximum(by2 - by1, 0.0)
        return bx1, by1, bx2, by2, ba

    top2_init = (jnp.full((_L,), _FNEG, jnp.float32), zeros_i,
                 jnp.full((_L,), _FNEG, jnp.float32), zeros_i)

    @plsc.parallel_loop(0, _NSLICE, unroll=8, carry=top2_init)
    def _pro(i, carry):
        sl = pl.ds(i * _L, _L)
        gsl = pl.ds(loff + i * _L, _L)
        v = s_v[sl]
        v = jnp.where(v > _SCORE_THR, v, _NEG)
        s_v[sl] = v
        area_v[sl] = (jnp.maximum(x2_v[gsl] - x1_v[gsl], 0.0)
                      * jnp.maximum(y2_v[gsl] - y1_v[gsl], 0.0))
        return _top2_update(carry, v, loff + i * _L + iota)

    _publish(_pro, 0)

    def _cond(carry):
        _, _, cont = carry
        return cont == 1

    def _round(carry):
        r, t, _ = carry
        pltpu.sync_copy(tbl_sh.at[r % 2], tab_v)
        m1 = plsc.load_gather(tab_v, [iota, zeros_i])
        i1 = plsc.bitcast(plsc.load_gather(tab_v, [iota, zeros_i + 1]),
                          jnp.int32)
        m2 = plsc.load_gather(tab_v, [iota, zeros_i + 2])
        i2 = plsc.bitcast(plsc.load_gather(tab_v, [iota, zeros_i + 3]),
                          jnp.int32)
        m1, i1, m2, i2 = _butterfly2((m1, i1, m2, i2))

        valid1 = m1 > (_NEG / 2.0)
        valid2 = m2 > (_NEG / 2.0)
        ax1, ay1, ax2, ay2, aa = _box(i1)
        bx1, by1, bx2, by2, ba = _box(i2)

        iw = jnp.maximum(jnp.minimum(ax2, bx2) - jnp.maximum(ax1, bx1), 0.0)
        ih = jnp.maximum(jnp.minimum(ay2, by2) - jnp.maximum(ay1, by1), 0.0)
        inter = iw * ih
        iou12 = inter / (aa + ba - inter + 1e-6)
        dual = (valid2 & jnp.logical_not(iou12 > _IOU_THR)
                & (jnp.full((_L,), t, jnp.int32) + 1 < _MAX_KEEP))

        m1_s = m1[0]
        d2_s = jnp.where(dual, 1, 0)[0]
        valid1_s = m1_s > (_NEG / 2.0)

        kept_v[t, :] = jnp.where(valid1, ax1, zf)
        kept_v[t + _MAX_KEEP, :] = jnp.where(valid1, ay1, zf)
        kept_v[t + 2 * _MAX_KEEP, :] = jnp.where(valid1, ax2, zf)
        kept_v[t + 3 * _MAX_KEEP, :] = jnp.where(valid1, ay2, zf)
        kept_v[t + 4 * _MAX_KEEP, :] = jnp.where(valid1, m1, zf)

        @pl.when(d2_s == 1)
        def _():
            kept_v[t + 1, :] = bx1
            kept_v[t + 1 + _MAX_KEEP, :] = by1
            kept_v[t + 1 + 2 * _MAX_KEEP, :] = bx2
            kept_v[t + 1 + 3 * _MAX_KEEP, :] = by2
            kept_v[t + 1 + 4 * _MAX_KEEP, :] = m2

        @plsc.parallel_loop(0, _NSLICE, unroll=2, carry=top2_init)
        def _pass(i, carry):
            sl = pl.ds(i * _L, _L)
            gsl = pl.ds(loff + i * _L, _L)
            idxv = loff + i * _L + iota
            v = s_v[sl]
            cx1 = x1_v[gsl]
            cy1 = y1_v[gsl]
            cx2 = x2_v[gsl]
            cy2 = y2_v[gsl]
            car = area_v[sl]
            iw1 = jnp.maximum(jnp.minimum(ax2, cx2) - jnp.maximum(ax1, cx1),
                              0.0)
            ih1 = jnp.maximum(jnp.minimum(ay2, cy2) - jnp.maximum(ay1, cy1),
                              0.0)
            in1 = iw1 * ih1
            iou1 = in1 / (aa + car - in1 + 1e-6)
            iw2 = jnp.maximum(jnp.minimum(bx2, cx2) - jnp.maximum(bx1, cx1),
                              0.0)
            ih2 = jnp.maximum(jnp.minimum(by2, cy2) - jnp.maximum(by1, cy1),
                              0.0)
            in2 = iw2 * ih2
            iou2 = in2 / (ba + car - in2 + 1e-6)
            s1 = (iou1 > _IOU_THR) & valid1
            s2 = (iou2 > _IOU_THR) & dual
            vn = jnp.where(s1 | s2, _NEG, v)
            s_v[sl] = vn
            return _top2_update(carry, vn, idxv)

        _publish(_pass, (r + 1) % 2)

        t_next = t + 1 + d2_s
        cont = jnp.where(valid1_s & (t_next < _MAX_KEEP), 1, 0)
        return r + 1, t_next, cont

    lax.while_loop(_cond, _round, (jnp.int32(0), jnp.int32(0), jnp.int32(1)))

    @pl.when(wid == 0)
    def _():
        pltpu.sync_copy(kept_v, out_h)


@jax.jit
def _nms_sc(x1, y1, x2, y2, s):
    mesh = plsc.VectorSubcoreMesh(core_axis_name="c", subcore_axis_name="s",
                                  num_cores=1)
    f = pl.kernel(
        _nms_body,
        out_type=jax.ShapeDtypeStruct((5 * _MAX_KEEP, _L), jnp.float32),
        mesh=mesh,
        compiler_params=pltpu.CompilerParams(needs_layout_passes=False,
                                             use_tc_tiling_on_sc=False),
        scratch_types=[
            pltpu.VMEM((_NPAD,), jnp.float32),
            pltpu.VMEM((_NPAD,), jnp.float32),
            pltpu.VMEM((_NPAD,), jnp.float32),
            pltpu.VMEM((_NPAD,), jnp.float32),
            pltpu.VMEM((_SHARD,), jnp.float32),
            pltpu.VMEM((_SHARD,), jnp.float32),
            pltpu.VMEM((5 * _MAX_KEEP, _L), jnp.float32),
            pltpu.VMEM((_NS, _L), jnp.float32),
            pltpu.VMEM((_L,), jnp.float32),
            pltpu.VMEM_SHARED((2, _NS, _L), jnp.float32),
        ],
    )
    return f(x1, y1, x2, y2, s)


def kernel(boxes, scores):
    pad = _NPAD - _N
    x1 = jnp.pad(boxes[:, 0], (0, pad))
    y1 = jnp.pad(boxes[:, 1], (0, pad))
    x2 = jnp.pad(boxes[:, 2], (0, pad))
    y2 = jnp.pad(boxes[:, 3], (0, pad))
    s = jnp.pad(scores, (0, pad))
    out = _nms_sc(x1, y1, x2, y2, s)
    return out[:, 0].reshape(5, _MAX_KEEP).T


# speedup vs baseline: 1.0037x; 1.0037x over previous
import jax
import jax.numpy as jnp
from jax import lax
from jax.experimental import pallas as pl
from jax.experimental.pallas import tpu as pltpu
from jax.experimental.pallas import tpu_sc as plsc

_N = 20000
_IOU_THR = 0.5
_SCORE_THR = 0.1
_MAX_KEEP = 256
_NEG = -1e10

_L = 16
_NS = 16
_NPAD = 20480
_SHARD = _NPAD // _NS
_NSLICE = _SHARD // _L
_FNEG = -3.0e38
_K = 4


def _nms_body(x1_h, y1_h, x2_h, y2_h, s_h, out_h,
              x1_v, y1_v, x2_v, y2_v, s_v, area_v, kept_v, tab_v, comm_v,
              tbl_sh):
    wid = lax.axis_index("s")
    loff = wid * _SHARD
    iota = lax.iota(jnp.int32, _L)
    zeros_i = jnp.zeros((_L,), jnp.int32)
    zf = jnp.zeros((_L,), jnp.float32)

    pltpu.sync_copy(x1_h, x1_v)
    pltpu.sync_copy(y1_h, y1_v)
    pltpu.sync_copy(x2_h, x2_v)
    pltpu.sync_copy(y2_h, y2_v)
    pltpu.sync_copy(s_h.at[pl.ds(loff, _SHARD)], s_v)

    @plsc.parallel_loop(0, 5 * _MAX_KEEP, unroll=8)
    def _zero(j):
        kept_v[j, :] = zf

    def _topk_update(carry, vn, idxv):
        av = [carry[2 * k] for k in range(_K)]
        ai = [carry[2 * k + 1] for k in range(_K)]
        g = [vn > av[k] for k in range(_K)]
        nv = [jnp.where(g[0], vn, av[0])]
        ni = [jnp.where(g[0], idxv, ai[0])]
        for k in range(1, _K):
            nv.append(jnp.where(g[k - 1], av[k - 1],
                                jnp.where(g[k], vn, av[k])))
            ni.append(jnp.where(g[k - 1], ai[k - 1],
                                jnp.where(g[k], idxv, ai[k])))
        out = []
        for k in range(_K):
            out += [nv[k], ni[k]]
        return tuple(out)

    def _better(xv, xi, yv, yi):
        return (yv > xv) | ((yv == xv) & (yi < xi))

    def _merge4(a, b):
        w = []
        for k in range(_K):
            av, ai = a[k]
            bv, bi = b[_K - 1 - k]
            p = _better(av, ai, bv, bi)
            w.append((jnp.where(p, bv, av), jnp.where(p, bi, ai)))

        def cas(i, j):
            pv, pi = w[i]
            qv, qi = w[j]
            p = _better(pv, pi, qv, qi)
            w[i] = (jnp.where(p, qv, pv), jnp.where(p, qi, pi))
            w[j] = (jnp.where(p, pv, qv), jnp.where(p, pi, qi))

        cas(0, 2)
        cas(1, 3)
        cas(0, 1)
        cas(2, 3)
        return w

    def _butterfly4(s):
        for sh in (8, 4, 2, 1):
            perm = iota ^ sh
            p = [(v.at[perm].get(mode="promise_in_bounds"),
                  i.at[perm].get(mode="promise_in_bounds")) for v, i in s]
            s = _merge4(s, p)
        return s

    def _publish(carry, slot):
        s = [(carry[2 * k], carry[2 * k + 1]) for k in range(_K)]
        s = _butterfly4(s)
        row = plsc.bitcast(s[_K - 1][1], jnp.float32)
        for k in range(_K - 1, -1, -1):
            v, i = s[k]
            fi = plsc.bitcast(i, jnp.float32)
            if k < _K - 1:
                row = jnp.where(iota == 2 * k + 1, fi, row)
            row = jnp.where(iota == 2 * k, v, row)
        comm_v[...] = row
        pltpu.sync_copy(comm_v, tbl_sh.at[slot, wid])
        plsc.subcore_barrier()

    def _box(idx_v):
        bx1 = plsc.load_gather(x1_v, [idx_v])
        by1 = plsc.load_gather(y1_v, [idx_v])
        bx2 = plsc.load_gather(x2_v, [idx_v])
        by2 = plsc.load_gather(y2_v, [idx_v])
        ba = jnp.maximum(bx2 - bx1, 0.0) * jnp.maximum(by2 - by1, 0.0)
        return (bx1, by1, bx2, by2, ba)

    def _iou(a, b):
        iw = jnp.maximum(jnp.minimum(a[2], b[2]) - jnp.maximum(a[0], b[0]),
                         0.0)
        ih = jnp.maximum(jnp.minimum(a[3], b[3]) - jnp.maximum(a[1], b[1]),
                         0.0)
        inter = iw * ih
        return inter / (a[4] + b[4] - inter + 1e-6)

    topk_init = tuple(x for _ in range(_K)
                      for x in (jnp.full((_L,), _FNEG, jnp.float32), zeros_i))

    @plsc.parallel_loop(0, _NSLICE, unroll=8, carry=topk_init)
    def _pro(i, carry):
        sl = pl.ds(i * _L, _L)
        gsl = pl.ds(loff + i * _L, _L)
        v = s_v[sl]
        v = jnp.where(v > _SCORE_THR, v, _NEG)
        s_v[sl] = v
        area_v[sl] = (jnp.maximum(x2_v[gsl] - x1_v[gsl], 0.0)
                      * jnp.maximum(y2_v[gsl] - y1_v[gsl], 0.0))
        return _topk_update(carry, v, loff + i * _L + iota)

    _publish(_pro, 0)

    def _cond(carry):
        _, _, cont = carry
        return cont == 1

    def _round(carry):
        r, t, _ = carry
        pltpu.sync_copy(tbl_sh.at[r % 2], tab_v)
        s = [(plsc.load_gather(tab_v, [iota, zeros_i + 2 * k]),
              plsc.bitcast(
                  plsc.load_gather(tab_v, [iota, zeros_i + 2 * k + 1]),
                  jnp.int32))
             for k in range(_K)]
        s = _butterfly4(s)
        vals = [v for v, _ in s]
        idxs = [i for _, i in s]
        valid = [v > (_NEG / 2.0) for v in vals]
        boxes = [_box(i) for i in idxs]

        tv = jnp.full((_L,), t, jnp.int32)
        c = [valid[0]]
        for k in range(1, _K):
            ck = c[k - 1] & valid[k] & (tv + k < _MAX_KEEP)
            for j in range(k):
                ck = ck & jnp.logical_not(_iou(boxes[j], boxes[k]) > _IOU_THR)
            c.append(ck)

        m1_s = vals[0][0]
        valid1_s = m1_s > (_NEG / 2.0)
        c_s = [jnp.where(ck, 1, 0)[0] for ck in c]

        kept_v[t, :] = jnp.where(valid[0], boxes[0][0], zf)
        kept_v[t + _MAX_KEEP, :] = jnp.where(valid[0], boxes[0][1], zf)
        kept_v[t + 2 * _MAX_KEEP, :] = jnp.where(valid[0], boxes[0][2], zf)
        kept_v[t + 3 * _MAX_KEEP, :] = jnp.where(valid[0], boxes[0][3], zf)
        kept_v[t + 4 * _MAX_KEEP, :] = jnp.where(valid[0], vals[0], zf)

        for k in range(1, _K):
            @pl.when(c_s[k] == 1)
            def _(k=k):
                kept_v[t + k, :] = boxes[k][0]
                kept_v[t + k + _MAX_KEEP, :] = boxes[k][1]
                kept_v[t + k + 2 * _MAX_KEEP, :] = boxes[k][2]
                kept_v[t + k + 3 * _MAX_KEEP, :] = boxes[k][3]
                kept_v[t + k + 4 * _MAX_KEEP, :] = vals[k]

        @plsc.parallel_loop(0, _NSLICE, unroll=4, carry=topk_init)
        def _pass(i, carry):
            sl = pl.ds(i * _L, _L)
            gsl = pl.ds(loff + i * _L, _L)
            idxv = loff + i * _L + iota
            v = s_v[sl]
            cb = (x1_v[gsl], y1_v[gsl], x2_v[gsl], y2_v[gsl], area_v[sl])
            supp = (_iou(boxes[0], cb) > _IOU_THR) & c[0]
            for k in range(1, _K):
                supp = supp | ((_iou(boxes[k], cb) > _IOU_THR) & c[k])
            vn = jnp.where(supp, _NEG, v)
            s_v[sl] = vn
            return _topk_update(carry, vn, idxv)

        _publish(_pass, (r + 1) % 2)

        t_next = t + 1 + c_s[1] + c_s[2] + c_s[3]
        cont = jnp.where(valid1_s & (t_next < _MAX_KEEP), 1, 0)
        return r + 1, t_next, cont

    lax.while_loop(_cond, _round, (jnp.int32(0), jnp.int32(0), jnp.int32(1)))

    @pl.when(wid == 0)
    def _():
        pltpu.sync_copy(kept_v, out_h)


@jax.jit
def _nms_sc(x1, y1, x2, y2, s):
    mesh = plsc.VectorSubcoreMesh(core_axis_name="c", subcore_axis_name="s",
                                  num_cores=1)
    f = pl.kernel(
        _nms_body,
        out_type=jax.ShapeDtypeStruct((5 * _MAX_KEEP, _L), jnp.float32),
        mesh=mesh,
        compiler_params=pltpu.CompilerParams(needs_layout_passes=False,
                                             use_tc_tiling_on_sc=False),
        scratch_types=[
            pltpu.VMEM((_NPAD,), jnp.float32),
            pltpu.VMEM((_NPAD,), jnp.float32),
            pltpu.VMEM((_NPAD,), jnp.float32),
            pltpu.VMEM((_NPAD,), jnp.float32),
            pltpu.VMEM((_SHARD,), jnp.float32),
            pltpu.VMEM((_SHARD,), jnp.float32),
            pltpu.VMEM((5 * _MAX_KEEP, _L), jnp.float32),
            pltpu.VMEM((_NS, _L), jnp.float32),
            pltpu.VMEM((_L,), jnp.float32),
            pltpu.VMEM_SHARED((2, _NS, _L), jnp.float32),
        ],
    )
    return f(x1, y1, x2, y2, s)


def kernel(boxes, scores):
    pad = _NPAD - _N
    x1 = jnp.pad(boxes[:, 0], (0, pad))
    y1 = jnp.pad(boxes[:, 1], (0, pad))
    x2 = jnp.pad(boxes[:, 2], (0, pad))
    y2 = jnp.pad(boxes[:, 3], (0, pad))
    s = jnp.pad(scores, (0, pad))
    out = _nms_sc(x1, y1, x2, y2, s)
    return out[:, 0].reshape(5, _MAX_KEEP).T


# speedup vs baseline: 1.0477x; 1.0438x over previous
import jax
import jax.numpy as jnp
from jax import lax
from jax.experimental import pallas as pl
from jax.experimental.pallas import tpu as pltpu
from jax.experimental.pallas import tpu_sc as plsc

_N = 20000
_IOU_THR = 0.5
_SCORE_THR = 0.1
_MAX_KEEP = 256
_NEG = -1e10

_L = 16
_NS = 16
_NPAD = 20480
_SHARD = _NPAD // _NS
_NSLICE = _SHARD // _L
_FNEG = -3.0e38
_K = 4


def _nms_body(x1_h, y1_h, x2_h, y2_h, s_h, out_h,
              x1_v, y1_v, x2_v, y2_v, s_v, area_v, kept_v, tab_v, comm_v,
              tbl_sh):
    wid = lax.axis_index("s")
    loff = wid * _SHARD
    iota = lax.iota(jnp.int32, _L)
    zeros_i = jnp.zeros((_L,), jnp.int32)
    zf = jnp.zeros((_L,), jnp.float32)

    pltpu.sync_copy(x1_h, x1_v)
    pltpu.sync_copy(y1_h, y1_v)
    pltpu.sync_copy(x2_h, x2_v)
    pltpu.sync_copy(y2_h, y2_v)
    pltpu.sync_copy(s_h.at[pl.ds(loff, _SHARD)], s_v)

    @plsc.parallel_loop(0, 5 * _MAX_KEEP, unroll=8)
    def _zero(j):
        kept_v[j, :] = zf

    def _topk_update(carry, vn, idxv):
        av = [carry[2 * k] for k in range(_K)]
        ai = [carry[2 * k + 1] for k in range(_K)]
        g = [vn > av[k] for k in range(_K)]
        nv = [jnp.where(g[0], vn, av[0])]
        ni = [jnp.where(g[0], idxv, ai[0])]
        for k in range(1, _K):
            nv.append(jnp.where(g[k - 1], av[k - 1],
                                jnp.where(g[k], vn, av[k])))
            ni.append(jnp.where(g[k - 1], ai[k - 1],
                                jnp.where(g[k], idxv, ai[k])))
        out = []
        for k in range(_K):
            out += [nv[k], ni[k]]
        return tuple(out)

    def _better(xv, xi, yv, yi):
        return (yv > xv) | ((yv == xv) & (yi < xi))

    def _merge4(a, b):
        w = []
        for k in range(_K):
            av, ai = a[k]
            bv, bi = b[_K - 1 - k]
            p = _better(av, ai, bv, bi)
            w.append((jnp.where(p, bv, av), jnp.where(p, bi, ai)))

        def cas(i, j):
            pv, pi = w[i]
            qv, qi = w[j]
            p = _better(pv, pi, qv, qi)
            w[i] = (jnp.where(p, qv, pv), jnp.where(p, qi, pi))
            w[j] = (jnp.where(p, pv, qv), jnp.where(p, pi, qi))

        cas(0, 2)
        cas(1, 3)
        cas(0, 1)
        cas(2, 3)
        return w

    def _butterfly4(s):
        for sh in (8, 4, 2, 1):
            perm = iota ^ sh
            p = [(v.at[perm].get(mode="promise_in_bounds"),
                  i.at[perm].get(mode="promise_in_bounds")) for v, i in s]
            s = _merge4(s, p)
        return s

    def _publish(carry, slot):
        s = [(carry[2 * k], carry[2 * k + 1]) for k in range(_K)]
        s = _butterfly4(s)
        row = plsc.bitcast(s[_K - 1][1], jnp.float32)
        for k in range(_K - 1, -1, -1):
            v, i = s[k]
            fi = plsc.bitcast(i, jnp.float32)
            if k < _K - 1:
                row = jnp.where(iota == 2 * k + 1, fi, row)
            row = jnp.where(iota == 2 * k, v, row)
        comm_v[...] = row
        pltpu.sync_copy(comm_v, tbl_sh.at[slot, wid])
        plsc.subcore_barrier()

    def _box(idx_v):
        bx1 = plsc.load_gather(x1_v, [idx_v])
        by1 = plsc.load_gather(y1_v, [idx_v])
        bx2 = plsc.load_gather(x2_v, [idx_v])
        by2 = plsc.load_gather(y2_v, [idx_v])
        ba = jnp.maximum(bx2 - bx1, 0.0) * jnp.maximum(by2 - by1, 0.0)
        return (bx1, by1, bx2, by2, ba)

    def _iou(a, b):
        iw = jnp.maximum(jnp.minimum(a[2], b[2]) - jnp.maximum(a[0], b[0]),
                         0.0)
        ih = jnp.maximum(jnp.minimum(a[3], b[3]) - jnp.maximum(a[1], b[1]),
                         0.0)
        inter = iw * ih
        return inter / (a[4] + b[4] - inter + 1e-6)

    topk_init = tuple(x for _ in range(_K)
                      for x in (jnp.full((_L,), _FNEG, jnp.float32), zeros_i))

    @plsc.parallel_loop(0, _NSLICE, unroll=8, carry=topk_init)
    def _pro(i, carry):
        sl = pl.ds(i * _L, _L)
        gsl = pl.ds(loff + i * _L, _L)
        v = s_v[sl]
        v = jnp.where(v > _SCORE_THR, v, _NEG)
        s_v[sl] = v
        area_v[sl] = (jnp.maximum(x2_v[gsl] - x1_v[gsl], 0.0)
                      * jnp.maximum(y2_v[gsl] - y1_v[gsl], 0.0))
        return _topk_update(carry, v, loff + i * _L + iota)

    _publish(_pro, 0)

    def _cond(carry):
        _, _, cont = carry
        return cont == 1

    def _round(carry):
        r, t, _ = carry
        pltpu.sync_copy(tbl_sh.at[r % 2], tab_v)
        s = [(plsc.load_gather(tab_v, [iota, zeros_i + 2 * k]),
              plsc.bitcast(
                  plsc.load_gather(tab_v, [iota, zeros_i + 2 * k + 1]),
                  jnp.int32))
             for k in range(_K)]
        s = _butterfly4(s)
        vals = [v for v, _ in s]
        idxs = [i for _, i in s]
        valid = [v > (_NEG / 2.0) for v in vals]
        boxes = [_box(i) for i in idxs]

        tv = jnp.full((_L,), t, jnp.int32)
        c = [valid[0]]
        for k in range(1, _K):
            ck = c[k - 1] & valid[k] & (tv + k < _MAX_KEEP)
            for j in range(k):
                ck = ck & jnp.logical_not(_iou(boxes[j], boxes[k]) > _IOU_THR)
            c.append(ck)

        m1_s = vals[0][0]
        valid1_s = m1_s > (_NEG / 2.0)
        c_s = [jnp.where(ck, 1, 0)[0] for ck in c]

        kept_v[t, :] = jnp.where(valid[0], boxes[0][0], zf)
        kept_v[t + _MAX_KEEP, :] = jnp.where(valid[0], boxes[0][1], zf)
        kept_v[t + 2 * _MAX_KEEP, :] = jnp.where(valid[0], boxes[0][2], zf)
        kept_v[t + 3 * _MAX_KEEP, :] = jnp.where(valid[0], boxes[0][3], zf)
        kept_v[t + 4 * _MAX_KEEP, :] = jnp.where(valid[0], vals[0], zf)

        for k in range(1, _K):
            @pl.when(c_s[k] == 1)
            def _(k=k):
                kept_v[t + k, :] = boxes[k][0]
                kept_v[t + k + _MAX_KEEP, :] = boxes[k][1]
                kept_v[t + k + 2 * _MAX_KEEP, :] = boxes[k][2]
                kept_v[t + k + 3 * _MAX_KEEP, :] = boxes[k][3]
                kept_v[t + k + 4 * _MAX_KEEP, :] = vals[k]

        @plsc.parallel_loop(0, _NSLICE, unroll=8, carry=topk_init)
        def _pass(i, carry):
            sl = pl.ds(i * _L, _L)
            gsl = pl.ds(loff + i * _L, _L)
            idxv = loff + i * _L + iota
            v = s_v[sl]
            cb = (x1_v[gsl], y1_v[gsl], x2_v[gsl], y2_v[gsl], area_v[sl])
            supp = (_iou(boxes[0], cb) > _IOU_THR) & c[0]
            for k in range(1, _K):
                supp = supp | ((_iou(boxes[k], cb) > _IOU_THR) & c[k])
            vn = jnp.where(supp, _NEG, v)
            s_v[sl] = vn
            return _topk_update(carry, vn, idxv)

        _publish(_pass, (r + 1) % 2)

        t_next = t + 1 + c_s[1] + c_s[2] + c_s[3]
        cont = jnp.where(valid1_s & (t_next < _MAX_KEEP), 1, 0)
        return r + 1, t_next, cont

    lax.while_loop(_cond, _round, (jnp.int32(0), jnp.int32(0), jnp.int32(1)))

    @pl.when(wid == 0)
    def _():
        pltpu.sync_copy(kept_v, out_h)


@jax.jit
def _nms_sc(x1, y1, x2, y2, s):
    mesh = plsc.VectorSubcoreMesh(core_axis_name="c", subcore_axis_name="s",
                                  num_cores=1)
    f = pl.kernel(
        _nms_body,
        out_type=jax.ShapeDtypeStruct((5 * _MAX_KEEP, _L), jnp.float32),
        mesh=mesh,
        compiler_params=pltpu.CompilerParams(needs_layout_passes=False,
                                             use_tc_tiling_on_sc=False),
        scratch_types=[
            pltpu.VMEM((_NPAD,), jnp.float32),
            pltpu.VMEM((_NPAD,), jnp.float32),
            pltpu.VMEM((_NPAD,), jnp.float32),
            pltpu.VMEM((_NPAD,), jnp.float32),
            pltpu.VMEM((_SHARD,), jnp.float32),
            pltpu.VMEM((_SHARD,), jnp.float32),
            pltpu.VMEM((5 * _MAX_KEEP, _L), jnp.float32),
            pltpu.VMEM((_NS, _L), jnp.float32),
            pltpu.VMEM((_L,), jnp.float32),
            pltpu.VMEM_SHARED((2, _NS, _L), jnp.float32),
        ],
    )
    return f(x1, y1, x2, y2, s)


def kernel(boxes, scores):
    pad = _NPAD - _N
    x1 = jnp.pad(boxes[:, 0], (0, pad))
    y1 = jnp.pad(boxes[:, 1], (0, pad))
    x2 = jnp.pad(boxes[:, 2], (0, pad))
    y2 = jnp.pad(boxes[:, 3], (0, pad))
    s = jnp.pad(scores, (0, pad))
    out = _nms_sc(x1, y1, x2, y2, s)
    return out[:, 0].reshape(5, _MAX_KEEP).T


# speedup vs baseline: 1.0623x; 1.0140x over previous
import jax
import jax.numpy as jnp
from jax import lax
from jax.experimental import pallas as pl
from jax.experimental.pallas import tpu as pltpu
from jax.experimental.pallas import tpu_sc as plsc

_N = 20000
_IOU_THR = 0.5
_SCORE_THR = 0.1
_MAX_KEEP = 256
_NEG = -1e10

_L = 16
_NS = 16
_NPAD = 20480
_SHARD = _NPAD // _NS
_NSLICE = _SHARD // _L
_FNEG = -3.0e38
_K = 4


def _nms_body(x1_h, y1_h, x2_h, y2_h, s_h, out_h,
              x1_v, y1_v, x2_v, y2_v, s_v, area_v, kept_v, tab_v, comm_v,
              tbl_sh):
    wid = lax.axis_index("s")
    loff = wid * _SHARD
    iota = lax.iota(jnp.int32, _L)
    zeros_i = jnp.zeros((_L,), jnp.int32)
    zf = jnp.zeros((_L,), jnp.float32)

    pltpu.sync_copy(x1_h, x1_v)
    pltpu.sync_copy(y1_h, y1_v)
    pltpu.sync_copy(x2_h, x2_v)
    pltpu.sync_copy(y2_h, y2_v)
    pltpu.sync_copy(s_h.at[pl.ds(loff, _SHARD)], s_v)

    @plsc.parallel_loop(0, 5 * _MAX_KEEP, unroll=8)
    def _zero(j):
        kept_v[j, :] = zf

    def _topk_update(carry, vn, idxv):
        av = [carry[2 * k] for k in range(_K)]
        ai = [carry[2 * k + 1] for k in range(_K)]
        g = [vn > av[k] for k in range(_K)]
        nv = [jnp.where(g[0], vn, av[0])]
        ni = [jnp.where(g[0], idxv, ai[0])]
        for k in range(1, _K):
            nv.append(jnp.where(g[k - 1], av[k - 1],
                                jnp.where(g[k], vn, av[k])))
            ni.append(jnp.where(g[k - 1], ai[k - 1],
                                jnp.where(g[k], idxv, ai[k])))
        out = []
        for k in range(_K):
            out += [nv[k], ni[k]]
        return tuple(out)

    def _better(xv, xi, yv, yi):
        return (yv > xv) | ((yv == xv) & (yi < xi))

    def _merge4(a, b):
        w = []
        for k in range(_K):
            av, ai = a[k]
            bv, bi = b[_K - 1 - k]
            p = _better(av, ai, bv, bi)
            w.append((jnp.where(p, bv, av), jnp.where(p, bi, ai)))

        def cas(i, j):
            pv, pi = w[i]
            qv, qi = w[j]
            p = _better(pv, pi, qv, qi)
            w[i] = (jnp.where(p, qv, pv), jnp.where(p, qi, pi))
            w[j] = (jnp.where(p, pv, qv), jnp.where(p, pi, qi))

        cas(0, 2)
        cas(1, 3)
        cas(0, 1)
        cas(2, 3)
        return w

    def _butterfly4(s):
        for sh in (8, 4, 2, 1):
            perm = iota ^ sh
            p = [(v.at[perm].get(mode="promise_in_bounds"),
                  i.at[perm].get(mode="promise_in_bounds")) for v, i in s]
            s = _merge4(s, p)
        return s

    def _publish(carry, slot):
        s = [(carry[2 * k], carry[2 * k + 1]) for k in range(_K)]
        s = _butterfly4(s)
        row = plsc.bitcast(s[_K - 1][1], jnp.float32)
        for k in range(_K - 1, -1, -1):
            v, i = s[k]
            fi = plsc.bitcast(i, jnp.float32)
            if k < _K - 1:
                row = jnp.where(iota == 2 * k + 1, fi, row)
            row = jnp.where(iota == 2 * k, v, row)
        comm_v[...] = row
        pltpu.sync_copy(comm_v, tbl_sh.at[slot, wid])
        plsc.subcore_barrier()

    def _box(idx_v):
        bx1 = plsc.load_gather(x1_v, [idx_v])
        by1 = plsc.load_gather(y1_v, [idx_v])
        bx2 = plsc.load_gather(x2_v, [idx_v])
        by2 = plsc.load_gather(y2_v, [idx_v])
        ba = jnp.maximum(bx2 - bx1, 0.0) * jnp.maximum(by2 - by1, 0.0)
        return (bx1, by1, bx2, by2, ba)

    def _iou(a, b):
        iw = jnp.maximum(jnp.minimum(a[2], b[2]) - jnp.maximum(a[0], b[0]),
                         0.0)
        ih = jnp.maximum(jnp.minimum(a[3], b[3]) - jnp.maximum(a[1], b[1]),
                         0.0)
        inter = iw * ih
        return inter / (a[4] + b[4] - inter + 1e-6)

    topk_init = tuple(x for _ in range(_K)
                      for x in (jnp.full((_L,), _FNEG, jnp.float32), zeros_i))

    @plsc.parallel_loop(0, _NSLICE, unroll=8, carry=topk_init)
    def _pro(i, carry):
        sl = pl.ds(i * _L, _L)
        gsl = pl.ds(loff + i * _L, _L)
        v = s_v[sl]
        v = jnp.where(v > _SCORE_THR, v, _NEG)
        s_v[sl] = v
        area_v[sl] = (jnp.maximum(x2_v[gsl] - x1_v[gsl], 0.0)
                      * jnp.maximum(y2_v[gsl] - y1_v[gsl], 0.0))
        return _topk_update(carry, v, loff + i * _L + iota)

    _publish(_pro, 0)

    def _cond(carry):
        _, _, cont = carry
        return cont == 1

    def _round(carry):
        r, t, _ = carry
        pltpu.sync_copy(tbl_sh.at[r % 2], tab_v)
        s = [(plsc.load_gather(tab_v, [iota, zeros_i + 2 * k]),
              plsc.bitcast(
                  plsc.load_gather(tab_v, [iota, zeros_i + 2 * k + 1]),
                  jnp.int32))
             for k in range(_K)]
        s = _butterfly4(s)
        vals = [v for v, _ in s]
        idxs = [i for _, i in s]
        valid = [v > (_NEG / 2.0) for v in vals]
        boxes = [_box(i) for i in idxs]

        tv = jnp.full((_L,), t, jnp.int32)
        c = [valid[0]]
        for k in range(1, _K):
            ck = c[k - 1] & valid[k] & (tv + k < _MAX_KEEP)
            for j in range(k):
                ck = ck & jnp.logical_not(_iou(boxes[j], boxes[k]) > _IOU_THR)
            c.append(ck)

        m1_s = vals[0][0]
        valid1_s = m1_s > (_NEG / 2.0)
        c_s = [jnp.where(ck, 1, 0)[0] for ck in c]

        kept_v[t, :] = jnp.where(valid[0], boxes[0][0], zf)
        kept_v[t + _MAX_KEEP, :] = jnp.where(valid[0], boxes[0][1], zf)
        kept_v[t + 2 * _MAX_KEEP, :] = jnp.where(valid[0], boxes[0][2], zf)
        kept_v[t + 3 * _MAX_KEEP, :] = jnp.where(valid[0], boxes[0][3], zf)
        kept_v[t + 4 * _MAX_KEEP, :] = jnp.where(valid[0], vals[0], zf)

        for k in range(1, _K):
            @pl.when(c_s[k] == 1)
            def _(k=k):
                kept_v[t + k, :] = boxes[k][0]
                kept_v[t + k + _MAX_KEEP, :] = boxes[k][1]
                kept_v[t + k + 2 * _MAX_KEEP, :] = boxes[k][2]
                kept_v[t + k + 3 * _MAX_KEEP, :] = boxes[k][3]
                kept_v[t + k + 4 * _MAX_KEEP, :] = vals[k]

        @plsc.parallel_loop(0, _NSLICE, unroll=16, carry=topk_init)
        def _pass(i, carry):
            sl = pl.ds(i * _L, _L)
            gsl = pl.ds(loff + i * _L, _L)
            idxv = loff + i * _L + iota
            v = s_v[sl]
            cb = (x1_v[gsl], y1_v[gsl], x2_v[gsl], y2_v[gsl], area_v[sl])
            supp = (_iou(boxes[0], cb) > _IOU_THR) & c[0]
            for k in range(1, _K):
                supp = supp | ((_iou(boxes[k], cb) > _IOU_THR) & c[k])
            vn = jnp.where(supp, _NEG, v)
            s_v[sl] = vn
            return _topk_update(carry, vn, idxv)

        _publish(_pass, (r + 1) % 2)

        t_next = t + 1 + c_s[1] + c_s[2] + c_s[3]
        cont = jnp.where(valid1_s & (t_next < _MAX_KEEP), 1, 0)
        return r + 1, t_next, cont

    lax.while_loop(_cond, _round, (jnp.int32(0), jnp.int32(0), jnp.int32(1)))

    @pl.when(wid == 0)
    def _():
        pltpu.sync_copy(kept_v, out_h)


@jax.jit
def _nms_sc(x1, y1, x2, y2, s):
    mesh = plsc.VectorSubcoreMesh(core_axis_name="c", subcore_axis_name="s",
                                  num_cores=1)
    f = pl.kernel(
        _nms_body,
        out_type=jax.ShapeDtypeStruct((5 * _MAX_KEEP, _L), jnp.float32),
        mesh=mesh,
        compiler_params=pltpu.CompilerParams(needs_layout_passes=False,
                                             use_tc_tiling_on_sc=False),
        scratch_types=[
            pltpu.VMEM((_NPAD,), jnp.float32),
            pltpu.VMEM((_NPAD,), jnp.float32),
            pltpu.VMEM((_NPAD,), jnp.float32),
            pltpu.VMEM((_NPAD,), jnp.float32),
            pltpu.VMEM((_SHARD,), jnp.float32),
            pltpu.VMEM((_SHARD,), jnp.float32),
            pltpu.VMEM((5 * _MAX_KEEP, _L), jnp.float32),
            pltpu.VMEM((_NS, _L), jnp.float32),
            pltpu.VMEM((_L,), jnp.float32),
            pltpu.VMEM_SHARED((2, _NS, _L), jnp.float32),
        ],
    )
    return f(x1, y1, x2, y2, s)


def kernel(boxes, scores):
    pad = _NPAD - _N
    x1 = jnp.pad(boxes[:, 0], (0, pad))
    y1 = jnp.pad(boxes[:, 1], (0, pad))
    x2 = jnp.pad(boxes[:, 2], (0, pad))
    y2 = jnp.pad(boxes[:, 3], (0, pad))
    s = jnp.pad(scores, (0, pad))
    out = _nms_sc(x1, y1, x2, y2, s)
    return out[:, 0].reshape(5, _MAX_KEEP).T


# speedup vs baseline: 1.0807x; 1.0173x over previous
import jax
import jax.numpy as jnp
from jax import lax
from jax.experimental import pallas as pl
from jax.experimental.pallas import tpu as pltpu
from jax.experimental.pallas import tpu_sc as plsc

_N = 20000
_IOU_THR = 0.5
_SCORE_THR = 0.1
_MAX_KEEP = 256
_NEG = -1e10

_L = 16
_NS = 16
_NPAD = 20480
_SHARD = _NPAD // _NS
_NSLICE = _SHARD // _L
_FNEG = -3.0e38
_K = 4


def _nms_body(x1_h, y1_h, x2_h, y2_h, s_h, out_h,
              x1_v, y1_v, x2_v, y2_v, s_v, area_v, kept_v, tab_v, comm_v,
              tbl_sh):
    wid = lax.axis_index("s")
    loff = wid * _SHARD
    iota = lax.iota(jnp.int32, _L)
    zeros_i = jnp.zeros((_L,), jnp.int32)
    zf = jnp.zeros((_L,), jnp.float32)

    pltpu.sync_copy(x1_h, x1_v)
    pltpu.sync_copy(y1_h, y1_v)
    pltpu.sync_copy(x2_h, x2_v)
    pltpu.sync_copy(y2_h, y2_v)
    pltpu.sync_copy(s_h.at[pl.ds(loff, _SHARD)], s_v)

    @plsc.parallel_loop(0, 5 * _MAX_KEEP, unroll=8)
    def _zero(j):
        kept_v[j, :] = zf

    def _topk_update(carry, vn, idxv):
        av = [carry[2 * k] for k in range(_K)]
        ai = [carry[2 * k + 1] for k in range(_K)]
        g = [vn > av[k] for k in range(_K)]
        nv = [jnp.where(g[0], vn, av[0])]
        ni = [jnp.where(g[0], idxv, ai[0])]
        for k in range(1, _K):
            nv.append(jnp.where(g[k - 1], av[k - 1],
                                jnp.where(g[k], vn, av[k])))
            ni.append(jnp.where(g[k - 1], ai[k - 1],
                                jnp.where(g[k], idxv, ai[k])))
        out = []
        for k in range(_K):
            out += [nv[k], ni[k]]
        return tuple(out)

    def _better(xv, xi, yv, yi):
        return (yv > xv) | ((yv == xv) & (yi < xi))

    def _merge4(a, b):
        w = []
        for k in range(_K):
            av, ai = a[k]
            bv, bi = b[_K - 1 - k]
            p = _better(av, ai, bv, bi)
            w.append((jnp.where(p, bv, av), jnp.where(p, bi, ai)))

        def cas(i, j):
            pv, pi = w[i]
            qv, qi = w[j]
            p = _better(pv, pi, qv, qi)
            w[i] = (jnp.where(p, qv, pv), jnp.where(p, qi, pi))
            w[j] = (jnp.where(p, pv, qv), jnp.where(p, pi, qi))

        cas(0, 2)
        cas(1, 3)
        cas(0, 1)
        cas(2, 3)
        return w

    def _butterfly4(s):
        for sh in (8, 4, 2, 1):
            perm = iota ^ sh
            p = [(v.at[perm].get(mode="promise_in_bounds"),
                  i.at[perm].get(mode="promise_in_bounds")) for v, i in s]
            s = _merge4(s, p)
        return s

    def _publish(carry, slot):
        s = [(carry[2 * k], carry[2 * k + 1]) for k in range(_K)]
        s = _butterfly4(s)
        row = plsc.bitcast(s[_K - 1][1], jnp.float32)
        for k in range(_K - 1, -1, -1):
            v, i = s[k]
            fi = plsc.bitcast(i, jnp.float32)
            if k < _K - 1:
                row = jnp.where(iota == 2 * k + 1, fi, row)
            row = jnp.where(iota == 2 * k, v, row)
        comm_v[...] = row
        pltpu.sync_copy(comm_v, tbl_sh.at[slot, wid])
        plsc.subcore_barrier()

    def _box(idx_v):
        bx1 = plsc.load_gather(x1_v, [idx_v])
        by1 = plsc.load_gather(y1_v, [idx_v])
        bx2 = plsc.load_gather(x2_v, [idx_v])
        by2 = plsc.load_gather(y2_v, [idx_v])
        ba = jnp.maximum(bx2 - bx1, 0.0) * jnp.maximum(by2 - by1, 0.0)
        return (bx1, by1, bx2, by2, ba)

    def _iou(a, b):
        iw = jnp.maximum(jnp.minimum(a[2], b[2]) - jnp.maximum(a[0], b[0]),
                         0.0)
        ih = jnp.maximum(jnp.minimum(a[3], b[3]) - jnp.maximum(a[1], b[1]),
                         0.0)
        inter = iw * ih
        return inter / (a[4] + b[4] - inter + 1e-6)

    topk_init = tuple(x for _ in range(_K)
                      for x in (jnp.full((_L,), _FNEG, jnp.float32), zeros_i))

    @plsc.parallel_loop(0, _NSLICE, unroll=8, carry=topk_init)
    def _pro(i, carry):
        sl = pl.ds(i * _L, _L)
        gsl = pl.ds(loff + i * _L, _L)
        v = s_v[sl]
        v = jnp.where(v > _SCORE_THR, v, _NEG)
        s_v[sl] = v
        area_v[sl] = (jnp.maximum(x2_v[gsl] - x1_v[gsl], 0.0)
                      * jnp.maximum(y2_v[gsl] - y1_v[gsl], 0.0))
        return _topk_update(carry, v, loff + i * _L + iota)

    _publish(_pro, 0)

    def _cond(carry):
        _, _, cont = carry
        return cont == 1

    def _round(carry):
        r, t, _ = carry
        pltpu.sync_copy(tbl_sh.at[r % 2], tab_v)
        s = [(plsc.load_gather(tab_v, [iota, zeros_i + 2 * k]),
              plsc.bitcast(
                  plsc.load_gather(tab_v, [iota, zeros_i + 2 * k + 1]),
                  jnp.int32))
             for k in range(_K)]
        s = _butterfly4(s)
        vals = [v for v, _ in s]
        idxs = [i for _, i in s]
        valid = [v > (_NEG / 2.0) for v in vals]
        boxes = [_box(i) for i in idxs]

        tv = jnp.full((_L,), t, jnp.int32)
        c = [valid[0]]
        for k in range(1, _K):
            ck = c[k - 1] & valid[k] & (tv + k < _MAX_KEEP)
            for j in range(k):
                ck = ck & jnp.logical_not(_iou(boxes[j], boxes[k]) > _IOU_THR)
            c.append(ck)

        m1_s = vals[0][0]
        valid1_s = m1_s > (_NEG / 2.0)
        c_s = [jnp.where(ck, 1, 0)[0] for ck in c]

        kept_v[t, :] = jnp.where(valid[0], boxes[0][0], zf)
        kept_v[t + _MAX_KEEP, :] = jnp.where(valid[0], boxes[0][1], zf)
        kept_v[t + 2 * _MAX_KEEP, :] = jnp.where(valid[0], boxes[0][2], zf)
        kept_v[t + 3 * _MAX_KEEP, :] = jnp.where(valid[0], boxes[0][3], zf)
        kept_v[t + 4 * _MAX_KEEP, :] = jnp.where(valid[0], vals[0], zf)

        for k in range(1, _K):
            @pl.when(c_s[k] == 1)
            def _(k=k):
                kept_v[t + k, :] = boxes[k][0]
                kept_v[t + k + _MAX_KEEP, :] = boxes[k][1]
                kept_v[t + k + 2 * _MAX_KEEP, :] = boxes[k][2]
                kept_v[t + k + 3 * _MAX_KEEP, :] = boxes[k][3]
                kept_v[t + k + 4 * _MAX_KEEP, :] = vals[k]

        @plsc.parallel_loop(0, _NSLICE, unroll=20, carry=topk_init)
        def _pass(i, carry):
            sl = pl.ds(i * _L, _L)
            gsl = pl.ds(loff + i * _L, _L)
            idxv = loff + i * _L + iota
            v = s_v[sl]
            cb = (x1_v[gsl], y1_v[gsl], x2_v[gsl], y2_v[gsl], area_v[sl])
            supp = (_iou(boxes[0], cb) > _IOU_THR) & c[0]
            for k in range(1, _K):
                supp = supp | ((_iou(boxes[k], cb) > _IOU_THR) & c[k])
            vn = jnp.where(supp, _NEG, v)
            s_v[sl] = vn
            return _topk_update(carry, vn, idxv)

        _publish(_pass, (r + 1) % 2)

        t_next = t + 1 + c_s[1] + c_s[2] + c_s[3]
        cont = jnp.where(valid1_s & (t_next < _MAX_KEEP), 1, 0)
        return r + 1, t_next, cont

    lax.while_loop(_cond, _round, (jnp.int32(0), jnp.int32(0), jnp.int32(1)))

    @pl.when(wid == 0)
    def _():
        pltpu.sync_copy(kept_v, out_h)


@jax.jit
def _nms_sc(x1, y1, x2, y2, s):
    mesh = plsc.VectorSubcoreMesh(core_axis_name="c", subcore_axis_name="s",
                                  num_cores=1)
    f = pl.kernel(
        _nms_body,
        out_type=jax.ShapeDtypeStruct((5 * _MAX_KEEP, _L), jnp.float32),
        mesh=mesh,
        compiler_params=pltpu.CompilerParams(needs_layout_passes=False,
                                             use_tc_tiling_on_sc=False),
        scratch_types=[
            pltpu.VMEM((_NPAD,), jnp.float32),
            pltpu.VMEM((_NPAD,), jnp.float32),
            pltpu.VMEM((_NPAD,), jnp.float32),
            pltpu.VMEM((_NPAD,), jnp.float32),
            pltpu.VMEM((_SHARD,), jnp.float32),
            pltpu.VMEM((_SHARD,), jnp.float32),
            pltpu.VMEM((5 * _MAX_KEEP, _L), jnp.float32),
            pltpu.VMEM((_NS, _L), jnp.float32),
            pltpu.VMEM((_L,), jnp.float32),
            pltpu.VMEM_SHARED((2, _NS, _L), jnp.float32),
        ],
    )
    return f(x1, y1, x2, y2, s)


def kernel(boxes, scores):
    pad = _NPAD - _N
    x1 = jnp.pad(boxes[:, 0], (0, pad))
    y1 = jnp.pad(boxes[:, 1], (0, pad))
    x2 = jnp.pad(boxes[:, 2], (0, pad))
    y2 = jnp.pad(boxes[:, 3], (0, pad))
    s = jnp.pad(scores, (0, pad))
    out = _nms_sc(x1, y1, x2, y2, s)
    return out[:, 0].reshape(5, _MAX_KEEP).T
